# Initial kernel scaffold; baseline (speedup 1.0000x reference)
import jax
import jax.numpy as jnp
from jax.experimental import pallas as pl


def kernel(inputs, table):
    raise NotImplementedError("write your pallas kernel here")


# speedup vs baseline: 1.1079x; 1.1079x over previous
import jax
import jax.numpy as jnp
from jax import lax
from jax.experimental import pallas as pl
from jax.experimental.pallas import tpu as pltpu
from jax.experimental.pallas import tpu_sc as plsc

NC = 2
NS = 16
NW = NC * NS

BATCH = 16384
HIST = 50
EMBED_DIM = 32
B_TOTAL = BATCH * HIST
B_PER_W = B_TOTAL // NW
CHUNK = 2560
N_CHUNKS = B_PER_W // CHUNK


def _gather_body(idx_hbm, table_hbm, out_hbm, idx_v, rows_v, sem):
    wid = lax.axis_index("s") * NC + lax.axis_index("c")
    base = wid * B_PER_W

    @pl.loop(0, N_CHUNKS)
    def _chunk(i):
        off = base + i * CHUNK
        pltpu.sync_copy(idx_hbm.at[pl.ds(off, CHUNK)], idx_v)
        pltpu.async_copy(table_hbm.at[idx_v], rows_v, sem).wait()
        pltpu.sync_copy(rows_v, out_hbm.at[pl.ds(off, CHUNK)])


def kernel(inputs, table):
    flat_idx = inputs.reshape(-1).astype(jnp.int32)
    mesh = plsc.VectorSubcoreMesh(
        core_axis_name="c", subcore_axis_name="s", num_cores=NC, num_subcores=NS
    )
    out = pl.kernel(
        _gather_body,
        out_type=jax.ShapeDtypeStruct((B_TOTAL, EMBED_DIM), jnp.float32),
        mesh=mesh,
        compiler_params=pltpu.CompilerParams(use_tc_tiling_on_sc=False),
        scratch_types=[
            pltpu.VMEM((CHUNK,), jnp.int32),
            pltpu.VMEM((CHUNK, EMBED_DIM), jnp.float32),
            pltpu.SemaphoreType.DMA,
        ],
    )(flat_idx, table)
    return out.reshape(BATCH, HIST, EMBED_DIM)


# speedup vs baseline: 1.1127x; 1.0043x over previous
import jax
import jax.numpy as jnp
from jax import lax
from jax.experimental import pallas as pl
from jax.experimental.pallas import tpu as pltpu
from jax.experimental.pallas import tpu_sc as plsc

NC = 2
NS = 16
NW = NC * NS

BATCH = 16384
HIST = 50
EMBED_DIM = 32
B_TOTAL = BATCH * HIST
B_PER_W = B_TOTAL // NW
CHUNK = 1280
N_CHUNKS = B_PER_W // CHUNK


def _gather_body(idx_hbm, table_hbm, out_hbm, idx_v, rows_a, rows_b, gs_a,
                 gs_b, os_a, os_b):
    wid = lax.axis_index("s") * NC + lax.axis_index("c")
    base = wid * B_PER_W
    pltpu.sync_copy(idx_hbm.at[pl.ds(base, B_PER_W)], idx_v)
    rows = (rows_a, rows_b)
    gsem = (gs_a, gs_b)
    osem = (os_a, os_b)
    g_h = [None, None]
    o_h = [None, None]
    for i in range(N_CHUNKS):
        s = i % 2
        if i >= 2:
            o_h[s].wait()
        g_h[s] = pltpu.async_copy(
            table_hbm.at[idx_v.at[pl.ds(i * CHUNK, CHUNK)]], rows[s], gsem[s])
        if i >= 1:
            p = (i - 1) % 2
            g_h[p].wait()
            o_h[p] = pltpu.async_copy(
                rows[p], out_hbm.at[pl.ds(base + (i - 1) * CHUNK, CHUNK)],
                osem[p])
    s = (N_CHUNKS - 1) % 2
    g_h[s].wait()
    o_h[s] = pltpu.async_copy(
        rows[s], out_hbm.at[pl.ds(base + (N_CHUNKS - 1) * CHUNK, CHUNK)],
        osem[s])
    o_h[1 - s].wait()
    o_h[s].wait()


def kernel(inputs, table):
    flat_idx = inputs.reshape(-1).astype(jnp.int32)
    mesh = plsc.VectorSubcoreMesh(
        core_axis_name="c", subcore_axis_name="s", num_cores=NC, num_subcores=NS
    )
    out = pl.kernel(
        _gather_body,
        out_type=jax.ShapeDtypeStruct((B_TOTAL, EMBED_DIM), jnp.float32),
        mesh=mesh,
        compiler_params=pltpu.CompilerParams(use_tc_tiling_on_sc=False),
        scratch_types=[
            pltpu.VMEM((B_PER_W,), jnp.int32),
            pltpu.VMEM((CHUNK, EMBED_DIM), jnp.float32),
            pltpu.VMEM((CHUNK, EMBED_DIM), jnp.float32),
            pltpu.SemaphoreType.DMA,
            pltpu.SemaphoreType.DMA,
            pltpu.SemaphoreType.DMA,
            pltpu.SemaphoreType.DMA,
        ],
    )(flat_idx, table)
    return out.reshape(BATCH, HIST, EMBED_DIM)


# speedup vs baseline: 1.8000x; 1.6178x over previous
import jax
import jax.numpy as jnp
from jax import lax
from jax.experimental import pallas as pl
from jax.experimental.pallas import tpu as pltpu
from jax.experimental.pallas import tpu_sc as plsc

NC = 2
NS = 16
NW = NC * NS

BATCH = 16384
HIST = 50
EMBED_DIM = 32
B_PER_W = BATCH // NW
CB = 8
N_CHUNKS = B_PER_W // CB


def _gather_body(idx_hbm, table_hbm, out_hbm, idx_v, rows_a, rows_b, gs_a,
                 gs_b, ws_a, ws_b):
    wid = lax.axis_index("s") * NC + lax.axis_index("c")
    base = wid * B_PER_W
    pltpu.sync_copy(idx_hbm.at[pl.ds(base, B_PER_W)], idx_v)
    rows = (rows_a, rows_b)
    gsem = (gs_a, gs_b)
    wsem = (ws_a, ws_b)

    def fire_gathers(c, buf):
        for j in range(CB):
            pltpu.async_copy(
                table_hbm.at[idx_v.at[c * CB + j]], rows[buf].at[j], gsem[buf])

    def drain_gathers(c, buf):
        pltpu.make_async_copy(
            out_hbm.at[pl.ds(base + c * CB, CB)], rows[buf], gsem[buf]).wait()

    def write_chunk(c, buf):
        pltpu.async_copy(
            rows[buf], out_hbm.at[pl.ds(base + c * CB, CB)], wsem[buf])

    def wait_write(c, buf):
        pltpu.make_async_copy(
            rows[buf], out_hbm.at[pl.ds(base + c * CB, CB)], wsem[buf]).wait()

    fire_gathers(0, 0)

    @pl.loop(0, N_CHUNKS, step=2)
    def _outer(c0):
        for b in (0, 1):
            c = c0 + b
            nxt = 1 - b

            @pl.when(c + 1 < N_CHUNKS)
            def _fire():
                @pl.when(c >= 1)
                def _w():
                    wait_write(c - 1, nxt)
                fire_gathers(c + 1, nxt)

            drain_gathers(c, b)
            write_chunk(c, b)

    wait_write(N_CHUNKS - 2, 0)
    wait_write(N_CHUNKS - 1, 1)


def kernel(inputs, table):
    idx = inputs.astype(jnp.int32)
    mesh = plsc.VectorSubcoreMesh(
        core_axis_name="c", subcore_axis_name="s", num_cores=NC, num_subcores=NS
    )
    out = pl.kernel(
        _gather_body,
        out_type=jax.ShapeDtypeStruct((BATCH, HIST, EMBED_DIM), jnp.float32),
        mesh=mesh,
        compiler_params=pltpu.CompilerParams(use_tc_tiling_on_sc=False),
        scratch_types=[
            pltpu.VMEM((B_PER_W, HIST), jnp.int32),
            pltpu.VMEM((CB, HIST, EMBED_DIM), jnp.float32),
            pltpu.VMEM((CB, HIST, EMBED_DIM), jnp.float32),
            pltpu.SemaphoreType.DMA,
            pltpu.SemaphoreType.DMA,
            pltpu.SemaphoreType.DMA,
            pltpu.SemaphoreType.DMA,
        ],
    )(idx, table)
    return out
